# baseline (speedup 1.0000x reference)
import jax
import jax.numpy as jnp
import numpy as np
from jax.experimental import pallas as pl
from jax.experimental.pallas import tpu as pltpu

C = 384
H = W = 224
N = H * W
HEADS = 12
HD = C // HEADS
NT = 28
TILES = NT * NT
NS = NT + 1
SLAB = 128

_INTERPRET = False


def _make_mask() -> np.ndarray:
    p = np.arange(64)
    e = np.arange(256)
    py, px = np.divmod(p, 8)
    half, rem = np.divmod(e, 128)
    ey, ix = np.divmod(rem, 8)
    ex = half * 8 + ix
    dy = ey[None, :] - py[:, None]
    dx = ex[None, :] - px[:, None]
    valid = (dy >= 0) & (dy <= 7) & (dx >= 0) & (dx <= 7)
    return np.where(valid, 0.0, -1e9).astype(np.float32)


_MASK = _make_mask()


def _fused_body(x0_ref, x1_ref, m_ref, wq_ref, wk_ref, wv_ref, wp_ref,
                bq_ref, bk_ref, bv_ref, bp_ref, o_ref,
                xs_scr, xq_scr, ks_scr, vs_scr, qs_scr):
    for txx in range(NS):
        p0 = x0_ref[:, pl.ds(8 * txx, 8), :].reshape(64, C)
        p1 = x1_ref[:, pl.ds(8 * txx, 8), :].reshape(64, C)
        xs_scr[pl.ds(SLAB * txx, 64)] = p0
        xs_scr[pl.ds(SLAB * txx + 64, 64)] = p1
    for tx in range(NT):
        c0 = x0_ref[4:8, pl.ds(8 * tx + 4, 8), :].reshape(32, C)
        c1 = x1_ref[0:4, pl.ds(8 * tx + 4, 8), :].reshape(32, C)
        xq_scr[pl.ds(64 * tx, 32)] = c0
        xq_scr[pl.ds(64 * tx + 32, 32)] = c1

    nt = (((1,), (1,)), ((), ()))
    ks_scr[...] = (jax.lax.dot_general(
        wk_ref[...], xs_scr[...], nt, preferred_element_type=jnp.float32)
        + bk_ref[...]).astype(jnp.bfloat16)
    vs_scr[...] = (jax.lax.dot_general(
        wv_ref[...], xs_scr[...], nt, preferred_element_type=jnp.float32)
        + bv_ref[...]).astype(jnp.bfloat16)
    qs_scr[...] = (jax.lax.dot_general(
        wq_ref[...], xq_scr[...], nt, preferred_element_type=jnp.float32)
        + bq_ref[...]).astype(jnp.bfloat16)

    mask = m_ref[...]
    wp = wp_ref[...]
    bpv = bp_ref[...]
    for tx in range(NT):
        qt = qs_scr[:, pl.ds(64 * tx, 64)].reshape(HEADS, HD, 64)
        kt = ks_scr[:, pl.ds(SLAB * tx, 256)].reshape(HEADS, HD, 256)
        vt = vs_scr[:, pl.ds(SLAB * tx, 256)].reshape(HEADS, HD, 256)
        s = jax.lax.dot_general(qt, kt, (((1,), (1,)), ((0,), (0,))),
                                preferred_element_type=jnp.float32)
        e = jnp.exp(s + mask[None])
        denom = jnp.sum(e, axis=-1)
        eb = e.astype(jnp.bfloat16)
        ot = jax.lax.dot_general(vt, eb, (((2,), (2,)), ((0,), (0,))),
                                 preferred_element_type=jnp.float32)
        ot = ot / denom[:, None, :]
        ob = ot.reshape(C, 64).astype(jnp.bfloat16)
        pt = jax.lax.dot_general(wp, ob, (((1,), (0,)), ((), ())),
                                 preferred_element_type=jnp.float32)
        o_ref[tx] = (pt + bpv).astype(jnp.bfloat16)


@jax.jit
def kernel(vid, Wq, bq, Wk, bk, Wv, bv, Wp, bp):
    scale = HD ** (-0.5)
    xb = vid.reshape(C, N).astype(jnp.bfloat16)
    xt = xb.T.reshape(H, W, C)
    xp = jnp.pad(xt, ((4, 4), (4, 4), (0, 0)), mode='reflect')

    mask = jnp.asarray(_MASK)
    wq_b = (Wq * scale).astype(jnp.bfloat16)
    wk_b = Wk.astype(jnp.bfloat16)
    wv_b = Wv.astype(jnp.bfloat16)
    wp_b = Wp.astype(jnp.bfloat16)
    bq2 = (bq * scale).reshape(C, 1)
    bk2 = bk.reshape(C, 1)
    bv2 = bv.reshape(C, 1)
    bp2 = bp.reshape(C, 1)

    cst = lambda shape: pl.BlockSpec(shape, lambda j: tuple(0 for _ in shape))
    out = pl.pallas_call(
        _fused_body,
        grid=(NT,),
        in_specs=[
            pl.BlockSpec((8, 232, C), lambda j: (j, 0, 0)),
            pl.BlockSpec((8, 232, C), lambda j: (j + 1, 0, 0)),
            cst((64, 256)),
            cst((C, C)), cst((C, C)), cst((C, C)), cst((C, C)),
            cst((C, 1)), cst((C, 1)), cst((C, 1)), cst((C, 1)),
        ],
        out_specs=pl.BlockSpec((NT, C, 64), lambda j: (j, 0, 0)),
        out_shape=jax.ShapeDtypeStruct((TILES, C, 64), jnp.bfloat16),
        scratch_shapes=[
            pltpu.VMEM((NS * SLAB, C), jnp.bfloat16),
            pltpu.VMEM((NT * 64, C), jnp.bfloat16),
            pltpu.VMEM((C, NS * SLAB), jnp.bfloat16),
            pltpu.VMEM((C, NS * SLAB), jnp.bfloat16),
            pltpu.VMEM((C, NT * 64), jnp.bfloat16),
        ],
        interpret=_INTERPRET,
    )(xp, xp, mask, wq_b, wk_b, wv_b, wp_b, bq2, bk2, bv2, bp2)

    o4 = out.astype(jnp.float32).reshape(NT, NT, C, 8, 8)
    o4 = o4.transpose(2, 0, 3, 1, 4).reshape(C, H, W)
    return o4.reshape(1, 1, C, H, W)


# speedup vs baseline: 1.4093x; 1.4093x over previous
import jax
import jax.numpy as jnp
import numpy as np
from jax.experimental import pallas as pl
from jax.experimental.pallas import tpu as pltpu

C = 384
H = W = 224
N = H * W
HEADS = 12
HD = C // HEADS
NT = 28
TILES = NT * NT
NS = NT + 1
SLAB = 128
HDP = HD + 8
CP = HEADS * HDP

_INTERPRET = False


def _make_mask() -> np.ndarray:
    p = np.arange(64)
    e = np.arange(256)
    py, px = np.divmod(p, 8)
    half, rem = np.divmod(e, 128)
    ey, ix = np.divmod(rem, 8)
    ex = half * 8 + ix
    dy = ey[None, :] - py[:, None]
    dx = ex[None, :] - px[:, None]
    valid = (dy >= 0) & (dy <= 7) & (dx >= 0) & (dx <= 7)
    return np.where(valid, 0.0, -1e9).astype(np.float32)


_MASK = _make_mask()


def _fused_body(x0_ref, x1_ref, m_ref, wq_ref, wk_ref, wv_ref, wp_ref,
                bq_ref, bk_ref, bv_ref, bp_ref, o_ref,
                xs_scr, xq_scr, ks_scr, vs_scr, qs_scr, ob_scr):
    for txx in range(NS):
        p0 = x0_ref[:, pl.ds(8 * txx, 8), :].reshape(64, C)
        p1 = x1_ref[:, pl.ds(8 * txx, 8), :].reshape(64, C)
        xs_scr[pl.ds(SLAB * txx, 64)] = p0
        xs_scr[pl.ds(SLAB * txx + 64, 64)] = p1
    for tx in range(NT):
        c0 = x0_ref[4:8, pl.ds(8 * tx + 4, 8), :].reshape(32, C)
        c1 = x1_ref[0:4, pl.ds(8 * tx + 4, 8), :].reshape(32, C)
        xq_scr[pl.ds(64 * tx, 32)] = c0
        xq_scr[pl.ds(64 * tx + 32, 32)] = c1

    nt = (((1,), (1,)), ((), ()))
    ks_scr[...] = (jax.lax.dot_general(
        wk_ref[...], xs_scr[...], nt, preferred_element_type=jnp.float32)
        + bk_ref[...]).astype(jnp.bfloat16)
    vs_scr[...] = (jax.lax.dot_general(
        wv_ref[...], xs_scr[...], nt, preferred_element_type=jnp.float32)
        + bv_ref[...]).astype(jnp.bfloat16)
    qs_scr[...] = (jax.lax.dot_general(
        wq_ref[...], xq_scr[...], nt, preferred_element_type=jnp.float32)
        + bq_ref[...]).astype(jnp.bfloat16)

    mask = m_ref[...]
    for tx in range(NT):
        qt = qs_scr[:, pl.ds(64 * tx, 64)].reshape(HEADS, HD, 64)
        kt = ks_scr[:, pl.ds(SLAB * tx, 256)].reshape(HEADS, HD, 256)
        vt = vs_scr[:, pl.ds(SLAB * tx, 256)].reshape(HEADS, HDP, 256)
        s = jax.lax.dot_general(qt, kt, (((1,), (1,)), ((0,), (0,))),
                                preferred_element_type=jnp.float32)
        eb = jnp.exp(s + mask[None]).astype(jnp.bfloat16)
        ot = jax.lax.dot_general(vt, eb, (((2,), (2,)), ((0,), (0,))),
                                 preferred_element_type=jnp.float32)
        ot = ot[:, 0:HD, :] / ot[:, HD:HD + 1, :]
        ob_scr[:, pl.ds(64 * tx, 64)] = ot.reshape(C, 64).astype(jnp.bfloat16)

    pt = jax.lax.dot_general(wp_ref[...], ob_scr[...], (((1,), (0,)), ((), ())),
                             preferred_element_type=jnp.float32)
    o_ref[0] = (pt + bp_ref[...]).astype(jnp.bfloat16)


@jax.jit
def kernel(vid, Wq, bq, Wk, bk, Wv, bv, Wp, bp):
    scale = HD ** (-0.5)
    xb = vid.reshape(C, N).astype(jnp.bfloat16)
    xt = xb.T.reshape(H, W, C)
    xp = jnp.pad(xt, ((4, 4), (4, 4), (0, 0)), mode='reflect')

    mask = jnp.asarray(_MASK)
    wq_b = (Wq * scale).astype(jnp.bfloat16)
    wk_b = Wk.astype(jnp.bfloat16)
    wv_b = jnp.pad(Wv.reshape(HEADS, HD, C),
                   ((0, 0), (0, 8), (0, 0))).reshape(CP, C).astype(jnp.bfloat16)
    bv2 = jnp.concatenate(
        [bv.reshape(HEADS, HD), jnp.ones((HEADS, 1), jnp.float32),
         jnp.zeros((HEADS, 7), jnp.float32)], axis=1).reshape(CP, 1)
    wp_b = Wp.astype(jnp.bfloat16)
    bq2 = (bq * scale).reshape(C, 1)
    bk2 = bk.reshape(C, 1)
    bp2 = bp.reshape(C, 1)

    cst = lambda shape: pl.BlockSpec(shape, lambda j: tuple(0 for _ in shape))
    out = pl.pallas_call(
        _fused_body,
        grid=(NT,),
        in_specs=[
            pl.BlockSpec((8, 232, C), lambda j: (j, 0, 0)),
            pl.BlockSpec((8, 232, C), lambda j: (j + 1, 0, 0)),
            cst((64, 256)),
            cst((C, C)), cst((C, C)), cst((CP, C)), cst((C, C)),
            cst((C, 1)), cst((C, 1)), cst((CP, 1)), cst((C, 1)),
        ],
        out_specs=pl.BlockSpec((1, C, NT * 64), lambda j: (j, 0, 0)),
        out_shape=jax.ShapeDtypeStruct((NT, C, NT * 64), jnp.bfloat16),
        scratch_shapes=[
            pltpu.VMEM((NS * SLAB, C), jnp.bfloat16),
            pltpu.VMEM((NT * 64, C), jnp.bfloat16),
            pltpu.VMEM((C, NS * SLAB), jnp.bfloat16),
            pltpu.VMEM((CP, NS * SLAB), jnp.bfloat16),
            pltpu.VMEM((C, NT * 64), jnp.bfloat16),
            pltpu.VMEM((C, NT * 64), jnp.bfloat16),
        ],
        interpret=_INTERPRET,
    )(xp, xp, mask, wq_b, wk_b, wv_b, wp_b, bq2, bk2, bv2, bp2)

    o4 = out.astype(jnp.float32).reshape(NT, C, NT, 8, 8)
    o4 = o4.transpose(1, 0, 3, 2, 4).reshape(C, H, W)
    return o4.reshape(1, 1, C, H, W)
